# baseline (speedup 1.0000x reference)
import functools

import jax
import jax.numpy as jnp
from jax import lax
from jax.experimental import pallas as pl
from jax.experimental.pallas import tpu as pltpu
from jax.experimental.pallas import tpu_sc as plsc

PERIOD = 24
LANES = 16
UNROLL = 8


def _seasonality_body(plane, K, x_hbm, out_hbm, buf, acc, rsem, wsem):
    cid = lax.axis_index("c")
    sid = lax.axis_index("s")
    b = sid * 2 + cid

    n_slices = plane // LANES
    n_iters = n_slices // UNROLL
    inv_k = jnp.float32(1.0 / K)

    pltpu.make_async_copy(x_hbm.at[b, 0], buf.at[0], rsem).start()
    pltpu.make_async_copy(x_hbm.at[b, 1], buf.at[1], rsem).start()

    for k in range(K):
        slot = k % 2
        pltpu.make_async_copy(x_hbm.at[b, k], buf.at[slot], rsem).wait()

        if k == 0:

            @pl.loop(0, n_iters)
            def _init(i):
                for u in range(UNROLL):
                    s = pl.ds((i * UNROLL + u) * LANES, LANES)
                    acc[s] = buf[0, s]

        elif k < K - 1:

            @pl.loop(0, n_iters)
            def _accum(i):
                for u in range(UNROLL):
                    s = pl.ds((i * UNROLL + u) * LANES, LANES)
                    acc[s] = acc[s] + buf[slot, s]

        else:

            @pl.loop(0, n_iters)
            def _accum_scale(i):
                for u in range(UNROLL):
                    s = pl.ds((i * UNROLL + u) * LANES, LANES)
                    acc[s] = (acc[s] + buf[slot, s]) * inv_k

        if k + 2 < K:
            pltpu.make_async_copy(x_hbm.at[b, k + 2], buf.at[slot], rsem).start()

    writes = [pltpu.make_async_copy(acc, out_hbm.at[b, k], wsem) for k in range(K)]
    for w in writes:
        w.start()
    for w in writes:
        w.wait()


def kernel(x):
    B, C, H, W = x.shape
    K = C // PERIOD
    plane = PERIOD * H * W

    x3 = x.reshape(B, K, plane)
    mesh = plsc.VectorSubcoreMesh(core_axis_name="c", subcore_axis_name="s")

    sc_kernel = pl.kernel(
        functools.partial(_seasonality_body, plane, K),
        out_type=jax.ShapeDtypeStruct((B, K, plane), jnp.float32),
        mesh=mesh,
        scratch_types=[
            pltpu.VMEM((2, plane), jnp.float32),
            pltpu.VMEM((plane,), jnp.float32),
            pltpu.SemaphoreType.DMA,
            pltpu.SemaphoreType.DMA,
        ],
    )
    out = sc_kernel(x3)
    return out.reshape(B, C, H, W)


# speedup vs baseline: 1.0608x; 1.0608x over previous
import functools

import jax
import jax.numpy as jnp
from jax import lax
from jax.experimental import pallas as pl
from jax.experimental.pallas import tpu as pltpu
from jax.experimental.pallas import tpu_sc as plsc

PERIOD = 24
LANES = 16
UNROLL = 8


def _seasonality_body(plane, K, G, x_hbm, out_hbm, buf, acc, rsem, wsem):
    cid = lax.axis_index("c")
    sid = lax.axis_index("s")
    b = sid * 2 + cid

    n_slices = plane // LANES
    n_iters = n_slices // UNROLL
    n_grp = K // G
    inv_k = jnp.float32(1.0 / K)

    pltpu.make_async_copy(x_hbm.at[b, pl.ds(0, G)], buf.at[0], rsem).start()
    pltpu.make_async_copy(x_hbm.at[b, pl.ds(G, G)], buf.at[1], rsem).start()

    for g in range(n_grp):
        slot = g % 2
        pltpu.make_async_copy(
            x_hbm.at[b, pl.ds(g * G, G)], buf.at[slot], rsem
        ).wait()

        if g == 0:

            @pl.loop(0, n_iters)
            def _init(i):
                for u in range(UNROLL):
                    s = pl.ds((i * UNROLL + u) * LANES, LANES)
                    v = buf[0, 0, s]
                    for j in range(1, G):
                        v = v + buf[0, j, s]
                    acc[s] = v

        elif g < n_grp - 1:

            @pl.loop(0, n_iters)
            def _accum(i):
                for u in range(UNROLL):
                    s = pl.ds((i * UNROLL + u) * LANES, LANES)
                    v = acc[s] + buf[slot, 0, s]
                    for j in range(1, G):
                        v = v + buf[slot, j, s]
                    acc[s] = v

        else:

            @pl.loop(0, n_iters)
            def _accum_scale(i):
                for u in range(UNROLL):
                    s = pl.ds((i * UNROLL + u) * LANES, LANES)
                    v = acc[s] + buf[slot, 0, s]
                    for j in range(1, G):
                        v = v + buf[slot, j, s]
                    acc[s] = v * inv_k

        if g + 2 < n_grp:
            pltpu.make_async_copy(
                x_hbm.at[b, pl.ds((g + 2) * G, G)], buf.at[slot], rsem
            ).start()

    writes = [pltpu.make_async_copy(acc, out_hbm.at[b, k], wsem) for k in range(K)]
    for w in writes:
        w.start()
    for w in writes:
        w.wait()


def kernel(x):
    B, C, H, W = x.shape
    K = C // PERIOD
    plane = PERIOD * H * W
    G = 2

    x3 = x.reshape(B, K, plane)
    mesh = plsc.VectorSubcoreMesh(core_axis_name="c", subcore_axis_name="s")

    sc_kernel = pl.kernel(
        functools.partial(_seasonality_body, plane, K, G),
        out_type=jax.ShapeDtypeStruct((B, K, plane), jnp.float32),
        mesh=mesh,
        scratch_types=[
            pltpu.VMEM((2, G, plane), jnp.float32),
            pltpu.VMEM((plane,), jnp.float32),
            pltpu.SemaphoreType.DMA,
            pltpu.SemaphoreType.DMA,
        ],
    )
    out = sc_kernel(x3)
    return out.reshape(B, C, H, W)
